# baseline (speedup 1.0000x reference)
import functools

import jax
import jax.numpy as jnp
from jax import lax
from jax.experimental import pallas as pl
from jax.experimental.pallas import tpu as pltpu
from jax.experimental.pallas import tpu_sc as plsc

B = 64
N = 64000
F = 512
S = 256
NB = N // S
NF = (N - F) // S + 1

NC = 2
NS = 16
NW = NC * NS
B_PER_W = B // NW

CF0 = 125
CF1 = NF - CF0
CB = CF0


@functools.partial(
    pl.kernel,
    mesh=plsc.VectorSubcoreMesh(core_axis_name="c", subcore_axis_name="s"),
    out_type=jax.ShapeDtypeStruct((B, NF, F), jnp.float32),
    compiler_params=pltpu.CompilerParams(
        use_tc_tiling_on_sc=False,
        disable_bounds_checks=True,
        disable_semaphore_checks=True,
    ),
    scratch_types=[
        pltpu.VMEM((3, CB + 1, S), jnp.float32),
        pltpu.SemaphoreType.DMA,
        pltpu.SemaphoreType.DMA,
    ],
)
def _frames_sc(sig_hbm, out_hbm, buf, sem_r, sem_w):
    wid = lax.axis_index("s") * NC + lax.axis_index("c")
    ntasks = B_PER_W * 2
    rh = [None] * 3
    wh = [None] * 3

    def task(t):
        return wid * B_PER_W + t // 2, t % 2

    def read(t, s):
        b, c = task(t)
        nf = CF0 if c == 0 else CF1
        return pltpu.async_copy(
            sig_hbm.at[b, pl.ds(c * CF0, nf + 1), :], buf.at[s, pl.ds(0, nf + 1), :],
            sem_r,
        )

    for t in range(3):
        rh[t] = read(t, t)
    for t in range(ntasks):
        s = t % 3
        b, c = task(t)
        nf = CF0 if c == 0 else CF1
        rh[s].wait()
        wh[s] = [
            pltpu.async_copy(
                buf.at[s, pl.ds(h, nf), :],
                out_hbm.at[b, pl.ds(c * CF0, nf), pl.ds(h * S, S)],
                sem_w,
            )
            for h in range(2)
        ]
        nt = t + 3
        if nt < ntasks:
            for w in wh[s]:
                w.wait()
            rh[s] = read(nt, s)
            wh[s] = None
    for ws in wh:
        if ws is not None:
            for w in ws:
                w.wait()


def kernel(sig):
    sig2 = sig.reshape(B, NB, S)
    out = _frames_sc(sig2)
    return out.reshape(B, 1, NF, F)


# speedup vs baseline: 1.0123x; 1.0123x over previous
import functools

import jax
import jax.numpy as jnp
from jax import lax
from jax.experimental import pallas as pl
from jax.experimental.pallas import tpu as pltpu
from jax.experimental.pallas import tpu_sc as plsc

B = 64
N = 64000
F = 512
S = 256
NB = N // S
NF = (N - F) // S + 1

NC = 2
NS = 16
NW = NC * NS
B_PER_W = B // NW

CF0 = 125
CF1 = NF - CF0
CB = CF0


@functools.partial(
    pl.kernel,
    mesh=plsc.VectorSubcoreMesh(core_axis_name="c", subcore_axis_name="s"),
    out_type=jax.ShapeDtypeStruct((B, NF, F), jnp.float32),
    compiler_params=pltpu.CompilerParams(
        use_tc_tiling_on_sc=False,
        disable_bounds_checks=True,
        disable_semaphore_checks=True,
    ),
    scratch_types=[
        pltpu.VMEM((B_PER_W, NB, S), jnp.float32),
        pltpu.SemaphoreType.DMA,
        pltpu.SemaphoreType.DMA,
    ],
)
def _frames_sc(sig_hbm, out_hbm, buf, sem_r, sem_w):
    wid = lax.axis_index("s") * NC + lax.axis_index("c")
    reads = [
        pltpu.async_copy(sig_hbm.at[wid * B_PER_W + t], buf.at[t], sem_r)
        for t in range(B_PER_W)
    ]
    writes = []
    for t in range(B_PER_W):
        b = wid * B_PER_W + t
        reads[t].wait()
        for h in range(2):
            writes.append(pltpu.async_copy(
                buf.at[t, pl.ds(h, NF), :],
                out_hbm.at[b, pl.ds(0, NF), pl.ds(h * S, S)],
                sem_w,
            ))
    for w in writes:
        w.wait()


def kernel(sig):
    sig2 = sig.reshape(B, NB, S)
    out = _frames_sc(sig2)
    return out.reshape(B, 1, NF, F)


# speedup vs baseline: 1.0153x; 1.0030x over previous
import functools

import jax
import jax.numpy as jnp
from jax import lax
from jax.experimental import pallas as pl
from jax.experimental.pallas import tpu as pltpu
from jax.experimental.pallas import tpu_sc as plsc

B = 64
N = 64000
F = 512
S = 256
NB = N // S
NF = (N - F) // S + 1

NC = 2
NS = 16
NW = NC * NS
B_PER_W = B // NW

CF0 = 125
CF1 = NF - CF0
CB = CF0


@functools.partial(
    pl.kernel,
    mesh=plsc.VectorSubcoreMesh(core_axis_name="c", subcore_axis_name="s"),
    out_type=jax.ShapeDtypeStruct((B, NF, F), jnp.float32),
    compiler_params=pltpu.CompilerParams(
        use_tc_tiling_on_sc=False,
        disable_bounds_checks=True,
        disable_semaphore_checks=True,
        skip_device_barrier=True,
    ),
    scratch_types=[
        pltpu.VMEM((B_PER_W, NB, S), jnp.float32),
        pltpu.SemaphoreType.DMA,
        pltpu.SemaphoreType.DMA,
    ],
)
def _frames_sc(sig_hbm, out_hbm, buf, sem_r, sem_w):
    wid = lax.axis_index("s") * NC + lax.axis_index("c")
    reads = [
        pltpu.async_copy(sig_hbm.at[wid * B_PER_W + t], buf.at[t], sem_r)
        for t in range(B_PER_W)
    ]
    writes = []
    for t in range(B_PER_W):
        b = wid * B_PER_W + t
        reads[t].wait()
        for h in range(2):
            writes.append(pltpu.async_copy(
                buf.at[t, pl.ds(h, NF), :],
                out_hbm.at[b, pl.ds(0, NF), pl.ds(h * S, S)],
                sem_w,
            ))
    for w in writes:
        w.wait()


def kernel(sig):
    sig2 = sig.reshape(B, NB, S)
    out = _frames_sc(sig2)
    return out.reshape(B, 1, NF, F)


# speedup vs baseline: 1.0224x; 1.0069x over previous
import functools

import jax
import jax.numpy as jnp
from jax import lax
from jax.experimental import pallas as pl
from jax.experimental.pallas import tpu as pltpu
from jax.experimental.pallas import tpu_sc as plsc

B = 64
N = 64000
F = 512
S = 256
NB = N // S
NF = (N - F) // S + 1

NC = 2
NS = 16
NW = NC * NS
B_PER_W = B // NW


@functools.partial(
    pl.kernel,
    mesh=plsc.VectorSubcoreMesh(core_axis_name="c", subcore_axis_name="s"),
    out_type=jax.ShapeDtypeStruct((B, NF, F), jnp.float32),
    compiler_params=pltpu.CompilerParams(
        use_tc_tiling_on_sc=False,
        disable_bounds_checks=True,
        disable_semaphore_checks=True,
    ),
    scratch_types=[
        pltpu.VMEM((B_PER_W, NB, S), jnp.float32),
        pltpu.SemaphoreType.DMA,
        pltpu.SemaphoreType.DMA,
    ],
)
def _frames_sc(sig_hbm, out_hbm, buf, sem_r, sem_w):
    wid = lax.axis_index("s") * NC + lax.axis_index("c")
    reads = [
        pltpu.async_copy(sig_hbm.at[wid * B_PER_W + t], buf.at[t], sem_r)
        for t in range(B_PER_W)
    ]
    writes = []
    for t in range(B_PER_W):
        b = wid * B_PER_W + t
        reads[t].wait()
        for h in range(2):
            writes.append(pltpu.async_copy(
                buf.at[t, pl.ds(h, NF), :],
                out_hbm.at[b, pl.ds(0, NF), pl.ds(h * S, S)],
                sem_w,
            ))
    for w in writes:
        w.wait()


def kernel(sig):
    sig2 = sig.reshape(B, NB, S)
    out = _frames_sc(sig2)
    return out.reshape(B, 1, NF, F)
